# Initial kernel scaffold; baseline (speedup 1.0000x reference)
import jax
import jax.numpy as jnp
from jax.experimental import pallas as pl


def kernel(user_table, item_table, buy_edges, click_edges, cart_edges, fav_edges):
    raise NotImplementedError("write your pallas kernel here")


# speedup vs baseline: 4.4933x; 4.4933x over previous
import jax
import jax.numpy as jnp
from jax import lax
from jax.experimental import pallas as pl
from jax.experimental.pallas import tpu as pltpu
from jax.experimental.pallas import tpu_sc as plsc

N = 50000
D = 128
QW = 32
NQ = 4
E = 150000
NS = 16
NC = 2
CHUNK = 128
CPT = 80
NCH = NS * CPT
EPAD = NCH * CHUNK
NACC = 50176
ZROWS = 784
FW = 400
NF = N // FW
ETYPES = 4
NCOMBO = 8

_MESH = plsc.VectorSubcoreMesh(core_axis_name="c", subcore_axis_name="s")


def _fill_zeros_2d(buf, rows):
    z = jnp.zeros((16,), jnp.float32)

    def body(i, _):
        buf[i, pl.ds(0, 16)] = z
        buf[i, pl.ds(16, 16)] = z
        return 0

    lax.fori_loop(0, rows, body, 0)


def _fill_const_1d(buf, n16, val):
    v = jnp.full((16,), val, jnp.float32)

    def body(i, _):
        buf[pl.ds(i * 16, 16)] = v
        return 0

    lax.fori_loop(0, n16, body, 0)


def _counts_body(dst_ref, recip_ref, cacc, didx_v, ones_v, zc_v):
    cid = lax.axis_index("c")
    tid = lax.axis_index("s")
    _fill_const_1d(ones_v, CHUNK // 16, 1.0)
    _fill_const_1d(zc_v, ZROWS // 16, 0.0)

    def zbody(k, _):
        def zz(z, _):
            pltpu.sync_copy(
                zc_v, cacc.at[k, pl.ds(tid * 3136 + z * ZROWS, ZROWS)])
            return 0

        lax.fori_loop(0, 4, zz, 0)
        return 0

    lax.fori_loop(0, 4, zbody, 0)
    plsc.subcore_barrier()

    def sbody(k, _):
        a = 4 * cid + k
        pltpu.sync_copy(dst_ref.at[a, pl.ds(tid * CPT, CPT)], didx_v)

        def inner(j, _):
            pltpu.sync_copy(ones_v, cacc.at[k].at[didx_v.at[j]], add=True)
            return 0

        lax.fori_loop(0, CPT, inner, 0)
        return 0

    lax.fori_loop(0, 4, sbody, 0)
    plsc.subcore_barrier()
    one = jnp.ones((16,), jnp.float32)

    def fbody(k, _):
        a = 4 * cid + k

        def fz(z, _):
            r0 = tid * 3136 + z * ZROWS
            pltpu.sync_copy(cacc.at[k, pl.ds(r0, ZROWS)], zc_v)

            def blk(i, _):
                c = zc_v[pl.ds(i * 16, 16)]
                zc_v[pl.ds(i * 16, 16)] = one / jnp.maximum(c, one)
                return 0

            lax.fori_loop(0, ZROWS // 16, blk, 0)
            pltpu.sync_copy(zc_v, recip_ref.at[a, pl.ds(r0, ZROWS)])
            return 0

        lax.fori_loop(0, 4, fz, 0)
        return 0

    lax.fori_loop(0, 4, fbody, 0)


ZSMALL = 196


def _one_pass(src_ref, sidx_ref, didx_ref, recip_ref, out_write, l1_ref,
              scratch):
    (acc, gsem0, gsem1) = scratch
    cid = lax.axis_index("c")
    tid = lax.axis_index("s")
    for r in range(2):
        q = 2 * cid + r

        def zero_phase(zbuf):
            _fill_zeros_2d(zbuf, ZSMALL)

            def zz(z, _):
                pltpu.sync_copy(
                    zbuf, acc.at[pl.ds(tid * 3136 + z * ZSMALL, ZSMALL)])
                return 0

            lax.fori_loop(0, 16, zz, 0)

        pl.run_scoped(zero_phase, pltpu.VMEM((ZSMALL, QW), jnp.float32))
        plsc.subcore_barrier()

        def scatter_phase(sidx_v, didx_v, rows0, rows1):
            pltpu.sync_copy(sidx_ref.at[pl.ds(tid * CPT, CPT)], sidx_v)
            pltpu.sync_copy(didx_ref.at[pl.ds(tid * CPT, CPT)], didx_v)
            pltpu.async_copy(src_ref.at[q].at[sidx_v.at[0]], rows0, gsem0)

            def body(jj, _):
                ca = 2 * jj
                cb = 2 * jj + 1
                pltpu.make_async_copy(
                    src_ref.at[q].at[sidx_v.at[ca]], rows0, gsem0).wait()
                pltpu.async_copy(
                    src_ref.at[q].at[sidx_v.at[cb]], rows1, gsem1)
                pltpu.sync_copy(rows0, acc.at[didx_v.at[ca]], add=True)
                pltpu.make_async_copy(
                    src_ref.at[q].at[sidx_v.at[cb]], rows1, gsem1).wait()

                @pl.when(cb + 1 < CPT)
                def _():
                    pltpu.async_copy(
                        src_ref.at[q].at[sidx_v.at[cb + 1]], rows0, gsem0)

                pltpu.sync_copy(rows1, acc.at[didx_v.at[cb]], add=True)
                return 0

            lax.fori_loop(0, CPT // 2, body, 0)

        pl.run_scoped(scatter_phase,
                      pltpu.VMEM((CPT, CHUNK), jnp.int32),
                      pltpu.VMEM((CPT, CHUNK), jnp.int32),
                      pltpu.VMEM((CHUNK, QW), jnp.float32),
                      pltpu.VMEM((CHUNK, QW), jnp.float32))
        plsc.subcore_barrier()

        def fin_phase(fbuf, rbuf, lbuf):
            half = jnp.full((16,), 0.5, jnp.float32)

            def kbody(k8, _):
                fk = tid + k8 * NS

                @pl.when(fk < NF)
                def _():
                    r0 = fk * FW
                    pltpu.sync_copy(acc.at[pl.ds(r0, FW)], fbuf)
                    pltpu.sync_copy(recip_ref.at[pl.ds(r0, FW)], rbuf)
                    if l1_ref is not None:
                        pltpu.sync_copy(l1_ref.at[q, pl.ds(r0, FW)], lbuf)

                    def fblk(b, _):
                        r16 = rbuf[pl.ds(b * 16, 16)]
                        for j in range(16):
                            v = jnp.full((16,), r16[j], jnp.float32)
                            row = b * 16 + j
                            for h in range(2):
                                sl = pl.ds(16 * h, 16)
                                x = fbuf[row, sl] * v
                                if l1_ref is not None:
                                    x = (x + lbuf[row, sl]) * half
                                fbuf[row, sl] = x
                        return 0

                    lax.fori_loop(0, FW // 16, fblk, 0)
                    out_write(q, r0, fbuf)

                return 0

            lax.fori_loop(0, 8, kbody, 0)

        pl.run_scoped(fin_phase,
                      pltpu.VMEM((FW, QW), jnp.float32),
                      pltpu.VMEM((FW,), jnp.float32),
                      pltpu.VMEM((FW, QW), jnp.float32))
        plsc.subcore_barrier()


def _pass_scratch():
    return [
        pltpu.VMEM_SHARED((NACC, QW), jnp.float32),
        pltpu.SemaphoreType.DMA,
        pltpu.SemaphoreType.DMA,
    ]


_SC_PARAMS = pltpu.CompilerParams(use_tc_tiling_on_sc=False)


def _make_layer_kernel(combos, n_in, out_type, name):
    n_out = len(out_type)

    def body(*refs):
        inputs = refs[:n_in]
        outputs = refs[n_in:n_in + n_out]
        scratch = refs[n_in + n_out:]
        for (src_i, sidx_i, didx_i, recip_i, l1_i, out_i, final) in combos:
            if final:
                def out_write(qs, r0, fb, out_ref=outputs[0], slot=out_i):
                    pltpu.sync_copy(
                        fb,
                        out_ref.at[slot, pl.ds(r0, FW), pl.ds(qs * QW, QW)])
            else:
                def out_write(qs, r0, fb, out_ref=outputs[out_i]):
                    pltpu.sync_copy(fb, out_ref.at[qs, pl.ds(r0, FW)])

            _one_pass(inputs[src_i], inputs[sidx_i], inputs[didx_i],
                      inputs[recip_i], out_write,
                      None if l1_i is None else inputs[l1_i], scratch)

    return pl.kernel(body, out_type=out_type, mesh=_MESH,
                     scratch_types=_pass_scratch(), name=name,
                     compiler_params=_SC_PARAMS)


def _pad_src(idx):
    pad = jnp.arange(EPAD - E, dtype=jnp.int32) % N
    return jnp.concatenate([idx, pad]).reshape(NCH, CHUNK)


def _pad_dst(idx):
    pad = N + (jnp.arange(EPAD - E, dtype=jnp.int32) % (NACC - N))
    return jnp.concatenate([idx, pad]).reshape(NCH, CHUNK)


def kernel(user_table, item_table, buy_edges, click_edges, cart_edges,
           fav_edges):
    edges = [buy_edges, click_edges, cart_edges, fav_edges]
    u_src, u_dst, i_src, i_dst = [], [], [], []
    for e in edges:
        u = e[0].astype(jnp.int32)
        i = e[1].astype(jnp.int32)
        u_src.append(_pad_src(u))
        u_dst.append(_pad_dst(u))
        i_src.append(_pad_src(i))
        i_dst.append(_pad_dst(i))

    user_q = user_table.reshape(N, NQ, QW).transpose(1, 0, 2)
    item_q = item_table.reshape(N, NQ, QW).transpose(1, 0, 2)

    counts_scratch = [
        pltpu.VMEM_SHARED((4, NACC), jnp.float32),
        pltpu.VMEM((CPT, CHUNK), jnp.int32),
        pltpu.VMEM((CHUNK,), jnp.float32),
        pltpu.VMEM((ZROWS,), jnp.float32),
    ]

    counts_k = pl.kernel(
        _counts_body,
        out_type=jax.ShapeDtypeStruct((NCOMBO, NACC), jnp.float32),
        mesh=_MESH, scratch_types=counts_scratch, name="gcn_counts",
        compiler_params=_SC_PARAMS)
    dst_stack = jnp.stack(
        [d for e in range(ETYPES) for d in (u_dst[e], i_dst[e])])
    recip_all = counts_k(dst_stack)
    recips = [recip_all[a] for a in range(NCOMBO)]

    l1_outs = []
    for half in range(2):
        ins = [user_q, item_q]
        combos = []
        oi = 0
        for e in (half * 2, half * 2 + 1):
            s_u = len(ins); ins.append(i_src[e])
            d_u = len(ins); ins.append(u_dst[e])
            r_u = len(ins); ins.append(recips[2 * e])
            combos.append((1, s_u, d_u, r_u, None, oi, False)); oi += 1
            s_i = len(ins); ins.append(u_src[e])
            d_i = len(ins); ins.append(i_dst[e])
            r_i = len(ins); ins.append(recips[2 * e + 1])
            combos.append((0, s_i, d_i, r_i, None, oi, False)); oi += 1
        out_type = [jax.ShapeDtypeStruct((NQ, N, QW), jnp.float32)
                    for _ in range(4)]
        k = _make_layer_kernel(combos, len(ins), out_type, f"gcn_l1_{half}")
        l1_outs += list(k(*ins))
    user1_q = [l1_outs[2 * e] for e in range(ETYPES)]
    item1_q = [l1_outs[2 * e + 1] for e in range(ETYPES)]

    halves = []
    for half in range(2):
        ins = []
        combos = []
        for k_e, e in enumerate((half * 2, half * 2 + 1)):
            src_u = len(ins); ins.append(item1_q[e])
            src_i = len(ins); ins.append(user1_q[e])
            s_u = len(ins); ins.append(i_src[e])
            d_u = len(ins); ins.append(u_dst[e])
            s_i = len(ins); ins.append(u_src[e])
            d_i = len(ins); ins.append(i_dst[e])
            r_u = len(ins); ins.append(recips[2 * e])
            r_i = len(ins); ins.append(recips[2 * e + 1])
            combos.append((src_u, s_u, d_u, r_u, src_i, 2 * k_e, True))
            combos.append((src_i, s_i, d_i, r_i, src_u, 2 * k_e + 1, True))
        out_type = [jax.ShapeDtypeStruct((4, N, D), jnp.float32)]
        k = _make_layer_kernel(combos, len(ins), out_type, f"gcn_l2_{half}")
        halves.append(k(*ins)[0])
    return jnp.concatenate(halves, axis=0)


# speedup vs baseline: 4.5796x; 1.0192x over previous
import jax
import jax.numpy as jnp
from jax import lax
from jax.experimental import pallas as pl
from jax.experimental.pallas import tpu as pltpu
from jax.experimental.pallas import tpu_sc as plsc

N = 50000
D = 128
QW = 32
NQ = 4
E = 150000
NS = 16
NC = 2
CHUNK = 128
CPT = 80
NCH = NS * CPT
EPAD = NCH * CHUNK
NACC = 50176
ZROWS = 784
FW = 80
NF = N // FW
NFR = (NF + NS - 1) // NS
ETYPES = 4
NCOMBO = 8

_MESH = plsc.VectorSubcoreMesh(core_axis_name="c", subcore_axis_name="s")


def _fill_zeros_2d(buf, rows):
    z = jnp.zeros((16,), jnp.float32)

    def body(i, _):
        buf[i, pl.ds(0, 16)] = z
        buf[i, pl.ds(16, 16)] = z
        return 0

    lax.fori_loop(0, rows, body, 0)


def _fill_const_1d(buf, n16, val):
    v = jnp.full((16,), val, jnp.float32)

    def body(i, _):
        buf[pl.ds(i * 16, 16)] = v
        return 0

    lax.fori_loop(0, n16, body, 0)


def _counts_body(dst_ref, recip_ref, cacc, didx_v, ones_v, zc_v):
    cid = lax.axis_index("c")
    tid = lax.axis_index("s")
    _fill_const_1d(ones_v, CHUNK // 16, 1.0)
    _fill_const_1d(zc_v, ZROWS // 16, 0.0)

    def zbody(k, _):
        def zz(z, _):
            pltpu.sync_copy(
                zc_v, cacc.at[k, pl.ds(tid * 3136 + z * ZROWS, ZROWS)])
            return 0

        lax.fori_loop(0, 4, zz, 0)
        return 0

    lax.fori_loop(0, 4, zbody, 0)
    plsc.subcore_barrier()

    def sbody(k, _):
        a = 4 * cid + k
        pltpu.sync_copy(dst_ref.at[a, pl.ds(tid * CPT, CPT)], didx_v)

        def inner(j, _):
            pltpu.sync_copy(ones_v, cacc.at[k].at[didx_v.at[j]], add=True)
            return 0

        lax.fori_loop(0, CPT, inner, 0)
        return 0

    lax.fori_loop(0, 4, sbody, 0)
    plsc.subcore_barrier()
    one = jnp.ones((16,), jnp.float32)

    def fbody(k, _):
        a = 4 * cid + k

        def fz(z, _):
            r0 = tid * 3136 + z * ZROWS
            pltpu.sync_copy(cacc.at[k, pl.ds(r0, ZROWS)], zc_v)

            def blk(i, _):
                c = zc_v[pl.ds(i * 16, 16)]
                zc_v[pl.ds(i * 16, 16)] = one / jnp.maximum(c, one)
                return 0

            lax.fori_loop(0, ZROWS // 16, blk, 0)
            pltpu.sync_copy(zc_v, recip_ref.at[a, pl.ds(r0, ZROWS)])
            return 0

        lax.fori_loop(0, 4, fz, 0)
        return 0

    lax.fori_loop(0, 4, fbody, 0)


ZSMALL = 196


RING = 4
LOOKAHEAD = 2
SUBB = 2
CPS = CPT // SUBB


def _one_pass(src_ref, sidx_ref, didx_ref, recip_ref, out_write, l1_ref,
              scratch):
    acc = scratch[0]
    gsems = scratch[1:1 + RING]
    ssems = scratch[1 + RING:1 + 2 * RING]
    cid = lax.axis_index("c")
    tid = lax.axis_index("s")
    for r in range(2):
        q = 2 * cid + r

        def zero_phase(zbuf):
            _fill_zeros_2d(zbuf, ZSMALL)

            def zz(z, _):
                pltpu.sync_copy(
                    zbuf, acc.at[pl.ds(tid * 3136 + z * ZSMALL, ZSMALL)])
                return 0

            lax.fori_loop(0, 16, zz, 0)

        pl.run_scoped(zero_phase, pltpu.VMEM((ZSMALL, QW), jnp.float32))
        plsc.subcore_barrier()

        for sub in range(SUBB):
            base = tid * CPT + sub * CPS

            def scatter_phase(sidx_v, didx_v, *bufs, base=base):
                pltpu.sync_copy(sidx_ref.at[pl.ds(base, CPS)], sidx_v)
                pltpu.sync_copy(didx_ref.at[pl.ds(base, CPS)], didx_v)
                for b in range(LOOKAHEAD):
                    pltpu.async_copy(
                        src_ref.at[q].at[sidx_v.at[b]], bufs[b], gsems[b])

                def body(o, _):
                    for b in range(RING):
                        idx = o * RING + b
                        pltpu.make_async_copy(
                            src_ref.at[q].at[sidx_v.at[idx]], bufs[b],
                            gsems[b]).wait()
                        pltpu.async_copy(
                            bufs[b], acc.at[didx_v.at[idx]], ssems[b],
                            add=True)
                        bb = (b + LOOKAHEAD) % RING

                        @pl.when(idx + LOOKAHEAD < CPS)
                        def _(idx=idx, b=b, bb=bb):
                            @pl.when(idx >= RING - LOOKAHEAD)
                            def _():
                                pltpu.make_async_copy(
                                    bufs[bb],
                                    acc.at[
                                        didx_v.at[idx + LOOKAHEAD - RING]],
                                    ssems[bb]).wait()

                            pltpu.async_copy(
                                src_ref.at[q].at[sidx_v.at[idx + LOOKAHEAD]],
                                bufs[bb], gsems[bb])

                    return 0

                lax.fori_loop(0, CPS // RING, body, 0)
                for b in range(RING):
                    pltpu.make_async_copy(
                        bufs[b], acc.at[didx_v.at[CPS - RING + b]],
                        ssems[b]).wait()

            pl.run_scoped(scatter_phase,
                          pltpu.VMEM((CPS, CHUNK), jnp.int32),
                          pltpu.VMEM((CPS, CHUNK), jnp.int32),
                          *[pltpu.VMEM((CHUNK, QW), jnp.float32)
                            for _ in range(RING)])
        plsc.subcore_barrier()

        def fin_phase(fbuf, rbuf, lbuf):
            half = jnp.full((16,), 0.5, jnp.float32)

            def kbody(k8, _):
                fk = tid + k8 * NS

                @pl.when(fk < NF)
                def _():
                    r0 = fk * FW
                    pltpu.sync_copy(acc.at[pl.ds(r0, FW)], fbuf)
                    pltpu.sync_copy(recip_ref.at[pl.ds(r0, FW)], rbuf)
                    if l1_ref is not None:
                        pltpu.sync_copy(l1_ref.at[q, pl.ds(r0, FW)], lbuf)

                    def fblk(b, _):
                        r16 = rbuf[pl.ds(b * 16, 16)]
                        for j in range(16):
                            v = jnp.full((16,), r16[j], jnp.float32)
                            row = b * 16 + j
                            for h in range(2):
                                sl = pl.ds(16 * h, 16)
                                x = fbuf[row, sl] * v
                                if l1_ref is not None:
                                    x = (x + lbuf[row, sl]) * half
                                fbuf[row, sl] = x
                        return 0

                    lax.fori_loop(0, FW // 16, fblk, 0)
                    out_write(q, r0, fbuf)

                return 0

            lax.fori_loop(0, NFR, kbody, 0)

        pl.run_scoped(fin_phase,
                      pltpu.VMEM((FW, QW), jnp.float32),
                      pltpu.VMEM((FW,), jnp.float32),
                      pltpu.VMEM((FW, QW), jnp.float32))
        plsc.subcore_barrier()


def _pass_scratch():
    return ([pltpu.VMEM_SHARED((NACC, QW), jnp.float32)]
            + [pltpu.SemaphoreType.DMA for _ in range(2 * RING)])


_SC_PARAMS = pltpu.CompilerParams(use_tc_tiling_on_sc=False)


def _make_layer_kernel(combos, n_in, out_type, name):
    n_out = len(out_type)

    def body(*refs):
        inputs = refs[:n_in]
        outputs = refs[n_in:n_in + n_out]
        scratch = refs[n_in + n_out:]
        for (src_i, sidx_i, didx_i, recip_i, l1_i, out_i, final) in combos:
            if final:
                def out_write(qs, r0, fb, out_ref=outputs[0], slot=out_i):
                    pltpu.sync_copy(
                        fb,
                        out_ref.at[slot, pl.ds(r0, FW), pl.ds(qs * QW, QW)])
            else:
                def out_write(qs, r0, fb, out_ref=outputs[out_i]):
                    pltpu.sync_copy(fb, out_ref.at[qs, pl.ds(r0, FW)])

            _one_pass(inputs[src_i], inputs[sidx_i], inputs[didx_i],
                      inputs[recip_i], out_write,
                      None if l1_i is None else inputs[l1_i], scratch)

    return pl.kernel(body, out_type=out_type, mesh=_MESH,
                     scratch_types=_pass_scratch(), name=name,
                     compiler_params=_SC_PARAMS)


def _pad_src(idx):
    pad = jnp.arange(EPAD - E, dtype=jnp.int32) % N
    return jnp.concatenate([idx, pad]).reshape(NCH, CHUNK)


def _pad_dst(idx):
    pad = N + (jnp.arange(EPAD - E, dtype=jnp.int32) % (NACC - N))
    return jnp.concatenate([idx, pad]).reshape(NCH, CHUNK)


def kernel(user_table, item_table, buy_edges, click_edges, cart_edges,
           fav_edges):
    edges = [buy_edges, click_edges, cart_edges, fav_edges]
    u_src, u_dst, i_src, i_dst = [], [], [], []
    for e in edges:
        u = e[0].astype(jnp.int32)
        i = e[1].astype(jnp.int32)
        u_src.append(_pad_src(u))
        u_dst.append(_pad_dst(u))
        i_src.append(_pad_src(i))
        i_dst.append(_pad_dst(i))

    user_q = user_table.reshape(N, NQ, QW).transpose(1, 0, 2)
    item_q = item_table.reshape(N, NQ, QW).transpose(1, 0, 2)

    counts_scratch = [
        pltpu.VMEM_SHARED((4, NACC), jnp.float32),
        pltpu.VMEM((CPT, CHUNK), jnp.int32),
        pltpu.VMEM((CHUNK,), jnp.float32),
        pltpu.VMEM((ZROWS,), jnp.float32),
    ]

    counts_k = pl.kernel(
        _counts_body,
        out_type=jax.ShapeDtypeStruct((NCOMBO, NACC), jnp.float32),
        mesh=_MESH, scratch_types=counts_scratch, name="gcn_counts",
        compiler_params=_SC_PARAMS)
    dst_stack = jnp.stack(
        [d for e in range(ETYPES) for d in (u_dst[e], i_dst[e])])
    recip_all = counts_k(dst_stack)
    recips = [recip_all[a] for a in range(NCOMBO)]

    ins = [user_q, item_q]
    combos = []
    oi = 0
    for e in range(ETYPES):
        s_u = len(ins); ins.append(i_src[e])
        d_u = len(ins); ins.append(u_dst[e])
        r_u = len(ins); ins.append(recips[2 * e])
        combos.append((1, s_u, d_u, r_u, None, oi, False)); oi += 1
        s_i = len(ins); ins.append(u_src[e])
        d_i = len(ins); ins.append(i_dst[e])
        r_i = len(ins); ins.append(recips[2 * e + 1])
        combos.append((0, s_i, d_i, r_i, None, oi, False)); oi += 1
    out_type = [jax.ShapeDtypeStruct((NQ, N, QW), jnp.float32)
                for _ in range(2 * ETYPES)]
    k1 = _make_layer_kernel(combos, len(ins), out_type, "gcn_l1")
    l1_outs = list(k1(*ins))
    user1_q = [l1_outs[2 * e] for e in range(ETYPES)]
    item1_q = [l1_outs[2 * e + 1] for e in range(ETYPES)]

    ins = []
    combos = []
    for e in range(ETYPES):
        src_u = len(ins); ins.append(item1_q[e])
        src_i = len(ins); ins.append(user1_q[e])
        s_u = len(ins); ins.append(i_src[e])
        d_u = len(ins); ins.append(u_dst[e])
        s_i = len(ins); ins.append(u_src[e])
        d_i = len(ins); ins.append(i_dst[e])
        r_u = len(ins); ins.append(recips[2 * e])
        r_i = len(ins); ins.append(recips[2 * e + 1])
        combos.append((src_u, s_u, d_u, r_u, src_i, 2 * e, True))
        combos.append((src_i, s_i, d_i, r_i, src_u, 2 * e + 1, True))
    out_type = [jax.ShapeDtypeStruct((2 * ETYPES, N, D), jnp.float32)]
    k2 = _make_layer_kernel(combos, len(ins), out_type, "gcn_l2")
    return k2(*ins)[0]


# speedup vs baseline: 4.5823x; 1.0006x over previous
import jax
import jax.numpy as jnp
from jax import lax
from jax.experimental import pallas as pl
from jax.experimental.pallas import tpu as pltpu
from jax.experimental.pallas import tpu_sc as plsc

N = 50000
D = 128
QW = 32
NQ = 4
E = 150000
NS = 16
NC = 2
CHUNK = 128
CPT = 80
NCH = NS * CPT
EPAD = NCH * CHUNK
NACC = 50176
ZROWS = 784
FW = 80
NF = N // FW
NFR = (NF + NS - 1) // NS
ETYPES = 4
NCOMBO = 8

_MESH = plsc.VectorSubcoreMesh(core_axis_name="c", subcore_axis_name="s")


def _fill_zeros_2d(buf, rows):
    z = jnp.zeros((16,), jnp.float32)

    def body(i, _):
        buf[i, pl.ds(0, 16)] = z
        buf[i, pl.ds(16, 16)] = z
        return 0

    lax.fori_loop(0, rows, body, 0)


def _fill_const_1d(buf, n16, val):
    v = jnp.full((16,), val, jnp.float32)

    def body(i, _):
        buf[pl.ds(i * 16, 16)] = v
        return 0

    lax.fori_loop(0, n16, body, 0)


def _counts_body(dst_ref, recip_ref, cacc, didx_v, ones_v, zc_v):
    cid = lax.axis_index("c")
    tid = lax.axis_index("s")
    _fill_const_1d(ones_v, CHUNK // 16, 1.0)
    _fill_const_1d(zc_v, ZROWS // 16, 0.0)

    def zbody(k, _):
        def zz(z, _):
            pltpu.sync_copy(
                zc_v, cacc.at[k, pl.ds(tid * 3136 + z * ZROWS, ZROWS)])
            return 0

        lax.fori_loop(0, 4, zz, 0)
        return 0

    lax.fori_loop(0, 4, zbody, 0)
    plsc.subcore_barrier()

    def sbody(k, _):
        a = 4 * cid + k
        pltpu.sync_copy(dst_ref.at[a, pl.ds(tid * CPT, CPT)], didx_v)

        def inner(j, _):
            pltpu.sync_copy(ones_v, cacc.at[k].at[didx_v.at[j]], add=True)
            return 0

        lax.fori_loop(0, CPT, inner, 0)
        return 0

    lax.fori_loop(0, 4, sbody, 0)
    plsc.subcore_barrier()
    one = jnp.ones((16,), jnp.float32)

    def fbody(k, _):
        a = 4 * cid + k

        def fz(z, _):
            r0 = tid * 3136 + z * ZROWS
            pltpu.sync_copy(cacc.at[k, pl.ds(r0, ZROWS)], zc_v)

            def blk(i, _):
                c = zc_v[pl.ds(i * 16, 16)]
                zc_v[pl.ds(i * 16, 16)] = one / jnp.maximum(c, one)
                return 0

            lax.fori_loop(0, ZROWS // 16, blk, 0)
            pltpu.sync_copy(zc_v, recip_ref.at[a, pl.ds(r0, ZROWS)])
            return 0

        lax.fori_loop(0, 4, fz, 0)
        return 0

    lax.fori_loop(0, 4, fbody, 0)


ZSMALL = 196


RING = 4
LOOKAHEAD = 2
SUBB = 2
CPS = CPT // SUBB


def _one_pass(src_ref, sidx_ref, didx_ref, recip_ref, out_write, l1_ref,
              scratch):
    acc = scratch[0]
    gsems = scratch[1:1 + RING]
    ssems = scratch[1 + RING:1 + 2 * RING]
    cid = lax.axis_index("c")
    tid = lax.axis_index("s")
    for r in range(2):
        q = 2 * cid + r

        def zero_phase(zbuf):
            _fill_zeros_2d(zbuf, ZSMALL)

            def zz(z, _):
                pltpu.sync_copy(
                    zbuf, acc.at[pl.ds(tid * 3136 + z * ZSMALL, ZSMALL)])
                return 0

            lax.fori_loop(0, 16, zz, 0)

        with jax.named_scope("ph_zero"):
            pl.run_scoped(zero_phase, pltpu.VMEM((ZSMALL, QW), jnp.float32))
            plsc.subcore_barrier()

        for sub in range(SUBB):
            base = tid * CPT + sub * CPS

            def scatter_phase(sidx_v, didx_v, *bufs, base=base):
                pltpu.sync_copy(sidx_ref.at[pl.ds(base, CPS)], sidx_v)
                pltpu.sync_copy(didx_ref.at[pl.ds(base, CPS)], didx_v)
                for b in range(LOOKAHEAD):
                    pltpu.async_copy(
                        src_ref.at[q].at[sidx_v.at[b]], bufs[b], gsems[b])

                def body(o, _):
                    for b in range(RING):
                        idx = o * RING + b
                        pltpu.make_async_copy(
                            src_ref.at[q].at[sidx_v.at[idx]], bufs[b],
                            gsems[b]).wait()
                        pltpu.async_copy(
                            bufs[b], acc.at[didx_v.at[idx]], ssems[b],
                            add=True)
                        bb = (b + LOOKAHEAD) % RING

                        @pl.when(idx + LOOKAHEAD < CPS)
                        def _(idx=idx, b=b, bb=bb):
                            @pl.when(idx >= RING - LOOKAHEAD)
                            def _():
                                pltpu.make_async_copy(
                                    bufs[bb],
                                    acc.at[
                                        didx_v.at[idx + LOOKAHEAD - RING]],
                                    ssems[bb]).wait()

                            pltpu.async_copy(
                                src_ref.at[q].at[sidx_v.at[idx + LOOKAHEAD]],
                                bufs[bb], gsems[bb])

                    return 0

                lax.fori_loop(0, CPS // RING, body, 0)
                for b in range(RING):
                    pltpu.make_async_copy(
                        bufs[b], acc.at[didx_v.at[CPS - RING + b]],
                        ssems[b]).wait()

            with jax.named_scope("ph_scat"):
                pl.run_scoped(scatter_phase,
                              pltpu.VMEM((CPS, CHUNK), jnp.int32),
                              pltpu.VMEM((CPS, CHUNK), jnp.int32),
                              *[pltpu.VMEM((CHUNK, QW), jnp.float32)
                                for _ in range(RING)])
        plsc.subcore_barrier()

        def fin_phase(fbuf, rbuf, lbuf):
            half = jnp.full((16,), 0.5, jnp.float32)

            def kbody(k8, _):
                fk = tid + k8 * NS

                @pl.when(fk < NF)
                def _():
                    r0 = fk * FW
                    pltpu.sync_copy(acc.at[pl.ds(r0, FW)], fbuf)
                    pltpu.sync_copy(recip_ref.at[pl.ds(r0, FW)], rbuf)
                    if l1_ref is not None:
                        pltpu.sync_copy(l1_ref.at[q, pl.ds(r0, FW)], lbuf)

                    def fblk(b, _):
                        r16 = rbuf[pl.ds(b * 16, 16)]
                        for j in range(16):
                            v = jnp.full((16,), r16[j], jnp.float32)
                            row = b * 16 + j
                            for h in range(2):
                                sl = pl.ds(16 * h, 16)
                                x = fbuf[row, sl] * v
                                if l1_ref is not None:
                                    x = (x + lbuf[row, sl]) * half
                                fbuf[row, sl] = x
                        return 0

                    lax.fori_loop(0, FW // 16, fblk, 0)
                    out_write(q, r0, fbuf)

                return 0

            lax.fori_loop(0, NFR, kbody, 0)

        with jax.named_scope("ph_fin"):
            pl.run_scoped(fin_phase,
                          pltpu.VMEM((FW, QW), jnp.float32),
                          pltpu.VMEM((FW,), jnp.float32),
                          pltpu.VMEM((FW, QW), jnp.float32))
            plsc.subcore_barrier()


def _pass_scratch():
    return ([pltpu.VMEM_SHARED((NACC, QW), jnp.float32)]
            + [pltpu.SemaphoreType.DMA for _ in range(2 * RING)])


_SC_PARAMS = pltpu.CompilerParams(use_tc_tiling_on_sc=False)


def _make_layer_kernel(combos, n_in, out_type, name):
    n_out = len(out_type)

    def body(*refs):
        inputs = refs[:n_in]
        outputs = refs[n_in:n_in + n_out]
        scratch = refs[n_in + n_out:]
        for (src_i, sidx_i, didx_i, recip_i, l1_i, out_i, final) in combos:
            if final:
                def out_write(qs, r0, fb, out_ref=outputs[0], slot=out_i):
                    pltpu.sync_copy(
                        fb,
                        out_ref.at[slot, pl.ds(r0, FW), pl.ds(qs * QW, QW)])
            else:
                def out_write(qs, r0, fb, out_ref=outputs[out_i]):
                    pltpu.sync_copy(fb, out_ref.at[qs, pl.ds(r0, FW)])

            _one_pass(inputs[src_i], inputs[sidx_i], inputs[didx_i],
                      inputs[recip_i], out_write,
                      None if l1_i is None else inputs[l1_i], scratch)

    return pl.kernel(body, out_type=out_type, mesh=_MESH,
                     scratch_types=_pass_scratch(), name=name,
                     compiler_params=_SC_PARAMS)


def _pad_src(idx):
    pad = jnp.arange(EPAD - E, dtype=jnp.int32) % N
    return jnp.concatenate([idx, pad]).reshape(NCH, CHUNK)


def _pad_dst(idx):
    pad = N + (jnp.arange(EPAD - E, dtype=jnp.int32) % (NACC - N))
    return jnp.concatenate([idx, pad]).reshape(NCH, CHUNK)


def kernel(user_table, item_table, buy_edges, click_edges, cart_edges,
           fav_edges):
    edges = [buy_edges, click_edges, cart_edges, fav_edges]
    u_src, u_dst, i_src, i_dst = [], [], [], []
    for e in edges:
        u = e[0].astype(jnp.int32)
        i = e[1].astype(jnp.int32)
        u_src.append(_pad_src(u))
        u_dst.append(_pad_dst(u))
        i_src.append(_pad_src(i))
        i_dst.append(_pad_dst(i))

    user_q = user_table.reshape(N, NQ, QW).transpose(1, 0, 2)
    item_q = item_table.reshape(N, NQ, QW).transpose(1, 0, 2)

    counts_scratch = [
        pltpu.VMEM_SHARED((4, NACC), jnp.float32),
        pltpu.VMEM((CPT, CHUNK), jnp.int32),
        pltpu.VMEM((CHUNK,), jnp.float32),
        pltpu.VMEM((ZROWS,), jnp.float32),
    ]

    counts_k = pl.kernel(
        _counts_body,
        out_type=jax.ShapeDtypeStruct((NCOMBO, NACC), jnp.float32),
        mesh=_MESH, scratch_types=counts_scratch, name="gcn_counts",
        compiler_params=_SC_PARAMS)
    dst_stack = jnp.stack(
        [d for e in range(ETYPES) for d in (u_dst[e], i_dst[e])])
    recip_all = counts_k(dst_stack)
    recips = [recip_all[a] for a in range(NCOMBO)]

    ins = [user_q, item_q]
    combos = []
    oi = 0
    for e in range(ETYPES):
        s_u = len(ins); ins.append(i_src[e])
        d_u = len(ins); ins.append(u_dst[e])
        r_u = len(ins); ins.append(recips[2 * e])
        combos.append((1, s_u, d_u, r_u, None, oi, False)); oi += 1
        s_i = len(ins); ins.append(u_src[e])
        d_i = len(ins); ins.append(i_dst[e])
        r_i = len(ins); ins.append(recips[2 * e + 1])
        combos.append((0, s_i, d_i, r_i, None, oi, False)); oi += 1
    out_type = [jax.ShapeDtypeStruct((NQ, N, QW), jnp.float32)
                for _ in range(2 * ETYPES)]
    k1 = _make_layer_kernel(combos, len(ins), out_type, "gcn_l1")
    l1_outs = list(k1(*ins))
    user1_q = [l1_outs[2 * e] for e in range(ETYPES)]
    item1_q = [l1_outs[2 * e + 1] for e in range(ETYPES)]

    ins = []
    combos = []
    for e in range(ETYPES):
        src_u = len(ins); ins.append(item1_q[e])
        src_i = len(ins); ins.append(user1_q[e])
        s_u = len(ins); ins.append(i_src[e])
        d_u = len(ins); ins.append(u_dst[e])
        s_i = len(ins); ins.append(u_src[e])
        d_i = len(ins); ins.append(i_dst[e])
        r_u = len(ins); ins.append(recips[2 * e])
        r_i = len(ins); ins.append(recips[2 * e + 1])
        combos.append((src_u, s_u, d_u, r_u, src_i, 2 * e, True))
        combos.append((src_i, s_i, d_i, r_i, src_u, 2 * e + 1, True))
    out_type = [jax.ShapeDtypeStruct((2 * ETYPES, N, D), jnp.float32)]
    k2 = _make_layer_kernel(combos, len(ins), out_type, "gcn_l2")
    return k2(*ins)[0]


# speedup vs baseline: 6.2739x; 1.3692x over previous
import jax
import jax.numpy as jnp
from jax import lax
from jax.experimental import pallas as pl
from jax.experimental.pallas import tpu as pltpu
from jax.experimental.pallas import tpu_sc as plsc

N = 50000
D = 128
QW = 32
NQ = 4
E = 150000
NS = 16
NC = 2
CHUNK = 128
CPT = 80
NCH = NS * CPT
EPAD = NCH * CHUNK
NACC = 50176
ZROWS = 784
FW = 400
NF = N // FW
NFR = (NF + NS - 1) // NS
ETYPES = 4
NCOMBO = 8

_MESH = plsc.VectorSubcoreMesh(core_axis_name="c", subcore_axis_name="s")


def _fill_zeros_2d(buf, rows):
    z = jnp.zeros((16,), jnp.float32)

    def body(i, _):
        buf[i, pl.ds(0, 16)] = z
        buf[i, pl.ds(16, 16)] = z
        return 0

    lax.fori_loop(0, rows, body, 0)


def _fill_const_1d(buf, n16, val):
    v = jnp.full((16,), val, jnp.float32)

    def body(i, _):
        buf[pl.ds(i * 16, 16)] = v
        return 0

    lax.fori_loop(0, n16, body, 0)


def _counts_body(dst_ref, recip_ref, cacc, didx_v, ones_v, zc_v):
    cid = lax.axis_index("c")
    tid = lax.axis_index("s")
    _fill_const_1d(ones_v, CHUNK // 16, 1.0)
    _fill_const_1d(zc_v, ZROWS // 16, 0.0)

    def zbody(k, _):
        def zz(z, _):
            pltpu.sync_copy(
                zc_v, cacc.at[k, pl.ds(tid * 3136 + z * ZROWS, ZROWS)])
            return 0

        lax.fori_loop(0, 4, zz, 0)
        return 0

    lax.fori_loop(0, 4, zbody, 0)
    plsc.subcore_barrier()

    def sbody(k, _):
        a = 4 * cid + k
        pltpu.sync_copy(dst_ref.at[a, pl.ds(tid * CPT, CPT)], didx_v)

        def inner(j, _):
            pltpu.sync_copy(ones_v, cacc.at[k].at[didx_v.at[j]], add=True)
            return 0

        lax.fori_loop(0, CPT, inner, 0)
        return 0

    lax.fori_loop(0, 4, sbody, 0)
    plsc.subcore_barrier()
    one = jnp.ones((16,), jnp.float32)

    def fbody(k, _):
        a = 4 * cid + k

        def fz(z, _):
            r0 = tid * 3136 + z * ZROWS
            pltpu.sync_copy(cacc.at[k, pl.ds(r0, ZROWS)], zc_v)

            def blk(i, _):
                c = zc_v[pl.ds(i * 16, 16)]
                zc_v[pl.ds(i * 16, 16)] = one / jnp.maximum(c, one)
                return 0

            lax.fori_loop(0, ZROWS // 16, blk, 0)
            pltpu.sync_copy(zc_v, recip_ref.at[a, pl.ds(r0, ZROWS)])
            return 0

        lax.fori_loop(0, 4, fz, 0)
        return 0

    lax.fori_loop(0, 4, fbody, 0)


ZSMALL = 196


RING = 4
LOOKAHEAD = 2
SUBB = 2
CPS = CPT // SUBB


def _one_pass(src_ref, sidx_ref, didx_ref, recip_ref, out_write, l1_ref,
              scratch):
    acc = scratch[0]
    gsems = scratch[1:1 + RING]
    ssems = scratch[1 + RING:1 + 2 * RING]
    cid = lax.axis_index("c")
    tid = lax.axis_index("s")
    for r in range(2):
        q = 2 * cid + r

        def zero_phase(zbuf):
            _fill_zeros_2d(zbuf, ZSMALL)

            def zz(z, _):
                pltpu.sync_copy(
                    zbuf, acc.at[pl.ds(tid * 3136 + z * ZSMALL, ZSMALL)])
                return 0

            lax.fori_loop(0, 16, zz, 0)

        with jax.named_scope("ph_zero"):
            pl.run_scoped(zero_phase, pltpu.VMEM((ZSMALL, QW), jnp.float32))
            plsc.subcore_barrier()

        for sub in range(SUBB):
            base = tid * CPT + sub * CPS

            def scatter_phase(sidx_v, didx_v, *bufs, base=base):
                pltpu.sync_copy(sidx_ref.at[pl.ds(base, CPS)], sidx_v)
                pltpu.sync_copy(didx_ref.at[pl.ds(base, CPS)], didx_v)
                for b in range(LOOKAHEAD):
                    pltpu.async_copy(
                        src_ref.at[q].at[sidx_v.at[b]], bufs[b], gsems[b])

                def body(o, _):
                    for b in range(RING):
                        idx = o * RING + b
                        pltpu.make_async_copy(
                            src_ref.at[q].at[sidx_v.at[idx]], bufs[b],
                            gsems[b]).wait()
                        pltpu.async_copy(
                            bufs[b], acc.at[didx_v.at[idx]], ssems[b],
                            add=True)
                        bb = (b + LOOKAHEAD) % RING

                        @pl.when(idx + LOOKAHEAD < CPS)
                        def _(idx=idx, b=b, bb=bb):
                            @pl.when(idx >= RING - LOOKAHEAD)
                            def _():
                                pltpu.make_async_copy(
                                    bufs[bb],
                                    acc.at[
                                        didx_v.at[idx + LOOKAHEAD - RING]],
                                    ssems[bb]).wait()

                            pltpu.async_copy(
                                src_ref.at[q].at[sidx_v.at[idx + LOOKAHEAD]],
                                bufs[bb], gsems[bb])

                    return 0

                lax.fori_loop(0, CPS // RING, body, 0)
                for b in range(RING):
                    pltpu.make_async_copy(
                        bufs[b], acc.at[didx_v.at[CPS - RING + b]],
                        ssems[b]).wait()

            with jax.named_scope("ph_scat"):
                pl.run_scoped(scatter_phase,
                              pltpu.VMEM((CPS, CHUNK), jnp.int32),
                              pltpu.VMEM((CPS, CHUNK), jnp.int32),
                              *[pltpu.VMEM((CHUNK, QW), jnp.float32)
                                for _ in range(RING)])
        plsc.subcore_barrier()

        def fin_phase(fbuf, rbuf, lbuf):
            half = jnp.full((16,), 0.5, jnp.float32)

            def kbody(k8, _):
                fk = tid + k8 * NS

                @pl.when(fk < NF)
                def _():
                    r0 = fk * FW
                    pltpu.sync_copy(acc.at[pl.ds(r0, FW)], fbuf)
                    pltpu.sync_copy(recip_ref.at[pl.ds(r0, FW)], rbuf)
                    if l1_ref is not None:
                        pltpu.sync_copy(l1_ref.at[q, pl.ds(r0, FW)], lbuf)

                    def fblk(b, _):
                        r16 = rbuf[pl.ds(b * 16, 16)]
                        for j in range(16):
                            v = jnp.full((16,), r16[j], jnp.float32)
                            row = b * 16 + j
                            for h in range(2):
                                sl = pl.ds(16 * h, 16)
                                x = fbuf[row, sl] * v
                                if l1_ref is not None:
                                    x = (x + lbuf[row, sl]) * half
                                fbuf[row, sl] = x
                        return 0

                    lax.fori_loop(0, FW // 16, fblk, 0)
                    out_write(q, r0, fbuf)

                return 0

            lax.fori_loop(0, NFR, kbody, 0)

        with jax.named_scope("ph_fin"):
            pl.run_scoped(fin_phase,
                          pltpu.VMEM((FW, QW), jnp.float32),
                          pltpu.VMEM((FW,), jnp.float32),
                          pltpu.VMEM((FW, QW), jnp.float32))
            plsc.subcore_barrier()


def _pass_scratch():
    return ([pltpu.VMEM_SHARED((NACC, QW), jnp.float32)]
            + [pltpu.SemaphoreType.DMA for _ in range(2 * RING)])


_SC_PARAMS = pltpu.CompilerParams(use_tc_tiling_on_sc=False)


def _make_layer_kernel(combos, n_in, out_type, name):
    n_out = len(out_type)

    def body(*refs):
        inputs = refs[:n_in]
        outputs = refs[n_in:n_in + n_out]
        scratch = refs[n_in + n_out:]
        for (src_i, sidx_i, didx_i, recip_i, l1_i, out_i, final) in combos:
            if final:
                def out_write(qs, r0, fb, out_ref=outputs[0], slot=out_i):
                    pltpu.sync_copy(
                        fb,
                        out_ref.at[slot, pl.ds(r0, FW), pl.ds(qs * QW, QW)])
            else:
                def out_write(qs, r0, fb, out_ref=outputs[out_i]):
                    pltpu.sync_copy(fb, out_ref.at[qs, pl.ds(r0, FW)])

            _one_pass(inputs[src_i], inputs[sidx_i], inputs[didx_i],
                      inputs[recip_i], out_write,
                      None if l1_i is None else inputs[l1_i], scratch)

    return pl.kernel(body, out_type=out_type, mesh=_MESH,
                     scratch_types=_pass_scratch(), name=name,
                     compiler_params=_SC_PARAMS)


def _pad_src(idx):
    pad = jnp.arange(EPAD - E, dtype=jnp.int32) % N
    return jnp.concatenate([idx, pad]).reshape(NCH, CHUNK)


def _pad_dst(idx):
    pad = N + (jnp.arange(EPAD - E, dtype=jnp.int32) % (NACC - N))
    return jnp.concatenate([idx, pad]).reshape(NCH, CHUNK)


def kernel(user_table, item_table, buy_edges, click_edges, cart_edges,
           fav_edges):
    edges = [buy_edges, click_edges, cart_edges, fav_edges]
    u_src, u_dst, i_src, i_dst = [], [], [], []
    for e in edges:
        u = e[0].astype(jnp.int32)
        i = e[1].astype(jnp.int32)
        u_src.append(_pad_src(u))
        u_dst.append(_pad_dst(u))
        i_src.append(_pad_src(i))
        i_dst.append(_pad_dst(i))

    user_q = user_table.reshape(N, NQ, QW).transpose(1, 0, 2)
    item_q = item_table.reshape(N, NQ, QW).transpose(1, 0, 2)

    counts_scratch = [
        pltpu.VMEM_SHARED((4, NACC), jnp.float32),
        pltpu.VMEM((CPT, CHUNK), jnp.int32),
        pltpu.VMEM((CHUNK,), jnp.float32),
        pltpu.VMEM((ZROWS,), jnp.float32),
    ]

    counts_k = pl.kernel(
        _counts_body,
        out_type=jax.ShapeDtypeStruct((NCOMBO, NACC), jnp.float32),
        mesh=_MESH, scratch_types=counts_scratch, name="gcn_counts",
        compiler_params=_SC_PARAMS)
    dst_stack = jnp.stack(
        [d for e in range(ETYPES) for d in (u_dst[e], i_dst[e])])
    recip_all = counts_k(dst_stack)
    recips = [recip_all[a] for a in range(NCOMBO)]

    ins = [user_q, item_q]
    combos = []
    oi = 0
    for e in range(ETYPES):
        s_u = len(ins); ins.append(i_src[e])
        d_u = len(ins); ins.append(u_dst[e])
        r_u = len(ins); ins.append(recips[2 * e])
        combos.append((1, s_u, d_u, r_u, None, oi, False)); oi += 1
        s_i = len(ins); ins.append(u_src[e])
        d_i = len(ins); ins.append(i_dst[e])
        r_i = len(ins); ins.append(recips[2 * e + 1])
        combos.append((0, s_i, d_i, r_i, None, oi, False)); oi += 1
    out_type = [jax.ShapeDtypeStruct((NQ, N, QW), jnp.float32)
                for _ in range(2 * ETYPES)]
    k1 = _make_layer_kernel(combos, len(ins), out_type, "gcn_l1")
    l1_outs = list(k1(*ins))
    user1_q = [l1_outs[2 * e] for e in range(ETYPES)]
    item1_q = [l1_outs[2 * e + 1] for e in range(ETYPES)]

    ins = []
    combos = []
    for e in range(ETYPES):
        src_u = len(ins); ins.append(item1_q[e])
        src_i = len(ins); ins.append(user1_q[e])
        s_u = len(ins); ins.append(i_src[e])
        d_u = len(ins); ins.append(u_dst[e])
        s_i = len(ins); ins.append(u_src[e])
        d_i = len(ins); ins.append(i_dst[e])
        r_u = len(ins); ins.append(recips[2 * e])
        r_i = len(ins); ins.append(recips[2 * e + 1])
        combos.append((src_u, s_u, d_u, r_u, src_i, 2 * e, True))
        combos.append((src_i, s_i, d_i, r_i, src_u, 2 * e + 1, True))
    out_type = [jax.ShapeDtypeStruct((2 * ETYPES, N, D), jnp.float32)]
    k2 = _make_layer_kernel(combos, len(ins), out_type, "gcn_l2")
    return k2(*ins)[0]


# speedup vs baseline: 6.9042x; 1.1005x over previous
import jax
import jax.numpy as jnp
from jax import lax
from jax.experimental import pallas as pl
from jax.experimental.pallas import tpu as pltpu
from jax.experimental.pallas import tpu_sc as plsc

N = 50000
D = 128
QW = 32
NQ = 4
E = 150000
NS = 16
NC = 2
CHUNK = 128
CPT = 80
NCH = NS * CPT
EPAD = NCH * CHUNK
NACC = 50176
ZROWS = 784
FW = 400
NF = N // FW
NFR = (NF + NS - 1) // NS
ETYPES = 4
NCOMBO = 8

_MESH = plsc.VectorSubcoreMesh(core_axis_name="c", subcore_axis_name="s")


def _fill_zeros_2d(buf, rows):
    z = jnp.zeros((16,), jnp.float32)

    def body(i, _):
        buf[i, pl.ds(0, 16)] = z
        buf[i, pl.ds(16, 16)] = z
        return 0

    lax.fori_loop(0, rows, body, 0)


def _fill_const_1d(buf, n16, val):
    v = jnp.full((16,), val, jnp.float32)

    def body(i, _):
        buf[pl.ds(i * 16, 16)] = v
        return 0

    lax.fori_loop(0, n16, body, 0)


def _counts_body(dst_ref, recip_ref, cacc, didx_v, ones_v, zc_v):
    cid = lax.axis_index("c")
    tid = lax.axis_index("s")
    _fill_const_1d(ones_v, CHUNK // 16, 1.0)
    _fill_const_1d(zc_v, ZROWS // 16, 0.0)

    def zbody(k, _):
        def zz(z, _):
            pltpu.sync_copy(
                zc_v, cacc.at[k, pl.ds(tid * 3136 + z * ZROWS, ZROWS)])
            return 0

        lax.fori_loop(0, 4, zz, 0)
        return 0

    lax.fori_loop(0, 4, zbody, 0)
    plsc.subcore_barrier()

    def sbody(k, _):
        a = 4 * cid + k
        pltpu.sync_copy(dst_ref.at[a, pl.ds(tid * CPT, CPT)], didx_v)

        def inner(j, _):
            pltpu.sync_copy(ones_v, cacc.at[k].at[didx_v.at[j]], add=True)
            return 0

        lax.fori_loop(0, CPT, inner, 0)
        return 0

    lax.fori_loop(0, 4, sbody, 0)
    plsc.subcore_barrier()
    one = jnp.ones((16,), jnp.float32)

    def fbody(k, _):
        a = 4 * cid + k

        def fz(z, _):
            r0 = tid * 3136 + z * ZROWS
            pltpu.sync_copy(cacc.at[k, pl.ds(r0, ZROWS)], zc_v)

            def blk(i, _):
                c = zc_v[pl.ds(i * 16, 16)]
                zc_v[pl.ds(i * 16, 16)] = one / jnp.maximum(c, one)
                return 0

            lax.fori_loop(0, ZROWS // 16, blk, 0)
            pltpu.sync_copy(zc_v, recip_ref.at[a, pl.ds(r0, ZROWS)])
            return 0

        lax.fori_loop(0, 4, fz, 0)
        return 0

    lax.fori_loop(0, 4, fbody, 0)


ZSMALL = 196


RING = 4
LOOKAHEAD = 2
SUBB = 2
CPS = CPT // SUBB


def _one_pass(src_ref, sidx_ref, didx_ref, recip_ref, out_write, l1_ref,
              scratch):
    acc = scratch[0]
    gsems = scratch[1:1 + RING]
    ssems = scratch[1 + RING:1 + 2 * RING]
    (asem, rsem, lsem) = scratch[1 + 2 * RING:4 + 2 * RING]
    cid = lax.axis_index("c")
    tid = lax.axis_index("s")
    for r in range(2):
        q = 2 * cid + r

        def zero_phase(zbuf):
            _fill_zeros_2d(zbuf, ZSMALL)

            def zz(z, _):
                pltpu.sync_copy(
                    zbuf, acc.at[pl.ds(tid * 3136 + z * ZSMALL, ZSMALL)])
                return 0

            lax.fori_loop(0, 16, zz, 0)

        with jax.named_scope("ph_zero"):
            pl.run_scoped(zero_phase, pltpu.VMEM((ZSMALL, QW), jnp.float32))
            plsc.subcore_barrier()

        for sub in range(SUBB):
            base = tid * CPT + sub * CPS

            def scatter_phase(sidx_v, didx_v, *bufs, base=base):
                pltpu.sync_copy(sidx_ref.at[pl.ds(base, CPS)], sidx_v)
                pltpu.sync_copy(didx_ref.at[pl.ds(base, CPS)], didx_v)
                for b in range(LOOKAHEAD):
                    pltpu.async_copy(
                        src_ref.at[q].at[sidx_v.at[b]], bufs[b], gsems[b])

                def body(o, _):
                    for b in range(RING):
                        idx = o * RING + b
                        pltpu.make_async_copy(
                            src_ref.at[q].at[sidx_v.at[idx]], bufs[b],
                            gsems[b]).wait()
                        pltpu.async_copy(
                            bufs[b], acc.at[didx_v.at[idx]], ssems[b],
                            add=True)
                        bb = (b + LOOKAHEAD) % RING

                        @pl.when(idx + LOOKAHEAD < CPS)
                        def _(idx=idx, b=b, bb=bb):
                            @pl.when(idx >= RING - LOOKAHEAD)
                            def _():
                                pltpu.make_async_copy(
                                    bufs[bb],
                                    acc.at[
                                        didx_v.at[idx + LOOKAHEAD - RING]],
                                    ssems[bb]).wait()

                            pltpu.async_copy(
                                src_ref.at[q].at[sidx_v.at[idx + LOOKAHEAD]],
                                bufs[bb], gsems[bb])

                    return 0

                lax.fori_loop(0, CPS // RING, body, 0)
                for b in range(RING):
                    pltpu.make_async_copy(
                        bufs[b], acc.at[didx_v.at[CPS - RING + b]],
                        ssems[b]).wait()

            with jax.named_scope("ph_scat"):
                pl.run_scoped(scatter_phase,
                              pltpu.VMEM((CPS, CHUNK), jnp.int32),
                              pltpu.VMEM((CPS, CHUNK), jnp.int32),
                              *[pltpu.VMEM((CHUNK, QW), jnp.float32)
                                for _ in range(RING)])
        plsc.subcore_barrier()

        def fin_phase(fbuf, rbuf, lbuf):
            half = jnp.full((16,), 0.5, jnp.float32)

            def kbody(k8, _):
                fk = tid + k8 * NS

                @pl.when(fk < NF)
                def _():
                    r0 = fk * FW
                    pltpu.async_copy(
                        recip_ref.at[pl.ds(r0, FW)], rbuf, rsem)
                    if l1_ref is not None:
                        pltpu.async_copy(
                            l1_ref.at[q, pl.ds(r0, FW)], lbuf, lsem)

                    @pl.when(k8 >= 1)
                    def _():
                        out_write(q, (fk - NS) * FW, fbuf, False)

                    pltpu.async_copy(acc.at[pl.ds(r0, FW)], fbuf, asem)
                    pltpu.make_async_copy(
                        recip_ref.at[pl.ds(r0, FW)], rbuf, rsem).wait()
                    if l1_ref is not None:
                        pltpu.make_async_copy(
                            l1_ref.at[q, pl.ds(r0, FW)], lbuf, lsem).wait()
                    pltpu.make_async_copy(
                        acc.at[pl.ds(r0, FW)], fbuf, asem).wait()

                    def fblk(b, _):
                        r16 = rbuf[pl.ds(b * 16, 16)]
                        for j in range(16):
                            v = jnp.full((16,), r16[j], jnp.float32)
                            row = b * 16 + j
                            for h in range(2):
                                sl = pl.ds(16 * h, 16)
                                x = fbuf[row, sl] * v
                                if l1_ref is not None:
                                    x = (x + lbuf[row, sl]) * half
                                fbuf[row, sl] = x
                        return 0

                    lax.fori_loop(0, FW // 16, fblk, 0)
                    out_write(q, r0, fbuf, True)

                return 0

            lax.fori_loop(0, NFR, kbody, 0)
            k_last = jnp.where(tid + (NFR - 1) * NS < NF, NFR - 1, NFR - 2)
            out_write(q, (tid + k_last * NS) * FW, fbuf, False)

        with jax.named_scope("ph_fin"):
            pl.run_scoped(fin_phase,
                          pltpu.VMEM((FW, QW), jnp.float32),
                          pltpu.VMEM((FW,), jnp.float32),
                          pltpu.VMEM((FW, QW), jnp.float32))
            plsc.subcore_barrier()


def _pass_scratch():
    return ([pltpu.VMEM_SHARED((NACC, QW), jnp.float32)]
            + [pltpu.SemaphoreType.DMA for _ in range(2 * RING + 4)])


_SC_PARAMS = pltpu.CompilerParams(use_tc_tiling_on_sc=False)


def _make_layer_kernel(combos, n_in, out_type, name):
    n_out = len(out_type)

    def body(*refs):
        inputs = refs[:n_in]
        outputs = refs[n_in:n_in + n_out]
        scratch = refs[n_in + n_out:]
        osem = scratch[4 + 2 * RING]
        for (src_i, sidx_i, didx_i, recip_i, l1_i, out_i, final) in combos:
            if final:
                def out_write(qs, r0, fb, start, out_ref=outputs[0],
                              slot=out_i):
                    dst = out_ref.at[slot, pl.ds(r0, FW),
                                     pl.ds(qs * QW, QW)]
                    if start:
                        pltpu.async_copy(fb, dst, osem)
                    else:
                        pltpu.make_async_copy(fb, dst, osem).wait()
            else:
                def out_write(qs, r0, fb, start, out_ref=outputs[out_i]):
                    dst = out_ref.at[qs, pl.ds(r0, FW)]
                    if start:
                        pltpu.async_copy(fb, dst, osem)
                    else:
                        pltpu.make_async_copy(fb, dst, osem).wait()

            _one_pass(inputs[src_i], inputs[sidx_i], inputs[didx_i],
                      inputs[recip_i], out_write,
                      None if l1_i is None else inputs[l1_i], scratch)

    return pl.kernel(body, out_type=out_type, mesh=_MESH,
                     scratch_types=_pass_scratch(), name=name,
                     compiler_params=_SC_PARAMS)


def _pad_src(idx):
    pad = jnp.arange(EPAD - E, dtype=jnp.int32) % N
    return jnp.concatenate([idx, pad]).reshape(NCH, CHUNK)


def _pad_dst(idx):
    pad = N + (jnp.arange(EPAD - E, dtype=jnp.int32) % (NACC - N))
    return jnp.concatenate([idx, pad]).reshape(NCH, CHUNK)


def kernel(user_table, item_table, buy_edges, click_edges, cart_edges,
           fav_edges):
    edges = [buy_edges, click_edges, cart_edges, fav_edges]
    u_src, u_dst, i_src, i_dst = [], [], [], []
    for e in edges:
        u = e[0].astype(jnp.int32)
        i = e[1].astype(jnp.int32)
        u_src.append(_pad_src(u))
        u_dst.append(_pad_dst(u))
        i_src.append(_pad_src(i))
        i_dst.append(_pad_dst(i))

    user_q = user_table.reshape(N, NQ, QW).transpose(1, 0, 2)
    item_q = item_table.reshape(N, NQ, QW).transpose(1, 0, 2)

    counts_scratch = [
        pltpu.VMEM_SHARED((4, NACC), jnp.float32),
        pltpu.VMEM((CPT, CHUNK), jnp.int32),
        pltpu.VMEM((CHUNK,), jnp.float32),
        pltpu.VMEM((ZROWS,), jnp.float32),
    ]

    counts_k = pl.kernel(
        _counts_body,
        out_type=jax.ShapeDtypeStruct((NCOMBO, NACC), jnp.float32),
        mesh=_MESH, scratch_types=counts_scratch, name="gcn_counts",
        compiler_params=_SC_PARAMS)
    dst_stack = jnp.stack(
        [d for e in range(ETYPES) for d in (u_dst[e], i_dst[e])])
    recip_all = counts_k(dst_stack)
    recips = [recip_all[a] for a in range(NCOMBO)]

    ins = [user_q, item_q]
    combos = []
    oi = 0
    for e in range(ETYPES):
        s_u = len(ins); ins.append(i_src[e])
        d_u = len(ins); ins.append(u_dst[e])
        r_u = len(ins); ins.append(recips[2 * e])
        combos.append((1, s_u, d_u, r_u, None, oi, False)); oi += 1
        s_i = len(ins); ins.append(u_src[e])
        d_i = len(ins); ins.append(i_dst[e])
        r_i = len(ins); ins.append(recips[2 * e + 1])
        combos.append((0, s_i, d_i, r_i, None, oi, False)); oi += 1
    out_type = [jax.ShapeDtypeStruct((NQ, N, QW), jnp.float32)
                for _ in range(2 * ETYPES)]
    k1 = _make_layer_kernel(combos, len(ins), out_type, "gcn_l1")
    l1_outs = list(k1(*ins))
    user1_q = [l1_outs[2 * e] for e in range(ETYPES)]
    item1_q = [l1_outs[2 * e + 1] for e in range(ETYPES)]

    ins = []
    combos = []
    for e in range(ETYPES):
        src_u = len(ins); ins.append(item1_q[e])
        src_i = len(ins); ins.append(user1_q[e])
        s_u = len(ins); ins.append(i_src[e])
        d_u = len(ins); ins.append(u_dst[e])
        s_i = len(ins); ins.append(u_src[e])
        d_i = len(ins); ins.append(i_dst[e])
        r_u = len(ins); ins.append(recips[2 * e])
        r_i = len(ins); ins.append(recips[2 * e + 1])
        combos.append((src_u, s_u, d_u, r_u, src_i, 2 * e, True))
        combos.append((src_i, s_i, d_i, r_i, src_u, 2 * e + 1, True))
    out_type = [jax.ShapeDtypeStruct((2 * ETYPES, N, D), jnp.float32)]
    k2 = _make_layer_kernel(combos, len(ins), out_type, "gcn_l2")
    return k2(*ins)[0]
